# baseline (speedup 1.0000x reference)
import functools

import jax
import jax.numpy as jnp
from jax import lax
from jax.experimental import pallas as pl
from jax.experimental.pallas import tpu as pltpu
from jax.experimental.pallas import tpu_sc as plsc

D_MODEL = 1024
BATCH = 16
LANES = 16
CHUNKS = D_MODEL // LANES


HALF = D_MODEL // 2


def _pool_body(hs_hbm, seq_hbm, out_hbm, seq_v, row_v, sem_a, sem_b):
    w = lax.axis_index("s")

    @pl.when(w < BATCH)
    def _():
        pltpu.sync_copy(seq_hbm, seq_v)
        seq = seq_v[...]
        idx = jnp.cumsum(seq) - 1
        lane = lax.iota(jnp.int32, 16)
        my_idx = jnp.sum(jnp.where(lane == w, idx, 0))
        row_src = hs_hbm.at[my_idx]
        cp_a = pltpu.async_copy(
            row_src.at[pl.ds(0, HALF)], row_v.at[pl.ds(0, HALF)], sem_a
        )
        cp_b = pltpu.async_copy(
            row_src.at[pl.ds(HALF, HALF)], row_v.at[pl.ds(HALF, HALF)], sem_b
        )

        def _ss_body(j, acc):
            base = j * (8 * LANES)
            for u in range(8):
                ch = row_v[pl.ds(base + u * LANES, LANES)]
                acc = acc + ch * ch
            return acc

        cp_a.wait()
        acc = lax.fori_loop(
            0, HALF // (8 * LANES), _ss_body, jnp.zeros((LANES,), jnp.float32)
        )
        cp_b.wait()
        acc = lax.fori_loop(HALF // (8 * LANES), CHUNKS // 8, _ss_body, acc)

        tot = jnp.maximum(jnp.sum(acc), 1e-24)
        t = jnp.full((LANES,), tot, dtype=jnp.float32)
        bits = lax.bitcast_convert_type(t, jnp.int32)
        bits = 0x5F3759DF - lax.shift_right_arithmetic(bits, 1)
        y = lax.bitcast_convert_type(bits, jnp.float32)
        for _ in range(3):
            y = y * (1.5 - 0.5 * t * y * y)

        def _scale_body(j, carry):
            base = j * (8 * LANES)
            for u in range(8):
                sl = pl.ds(base + u * LANES, LANES)
                row_v[sl] = row_v[sl] * y
            return carry

        out_row = out_hbm.at[w]
        lax.fori_loop(0, HALF // (8 * LANES), _scale_body, 0)
        st_a = pltpu.async_copy(
            row_v.at[pl.ds(0, HALF)], out_row.at[pl.ds(0, HALF)], sem_a
        )
        lax.fori_loop(HALF // (8 * LANES), CHUNKS // 8, _scale_body, 0)
        st_b = pltpu.async_copy(
            row_v.at[pl.ds(HALF, HALF)], out_row.at[pl.ds(HALF, HALF)], sem_b
        )
        st_a.wait()
        st_b.wait()


def kernel(hidden_states, extend_seq_lens):
    seq = extend_seq_lens.astype(jnp.int32)
    pooled = functools.partial(
        pl.kernel,
        mesh=plsc.VectorSubcoreMesh(
            core_axis_name="c", subcore_axis_name="s", num_cores=1
        ),
        out_type=jax.ShapeDtypeStruct((BATCH, D_MODEL), jnp.float32),
        scratch_types=[
            pltpu.VMEM((BATCH,), jnp.int32),
            pltpu.VMEM((D_MODEL,), jnp.float32),
            pltpu.SemaphoreType.DMA,
            pltpu.SemaphoreType.DMA,
        ],
        compiler_params=pltpu.CompilerParams(needs_layout_passes=False),
    )(_pool_body)(hidden_states, seq)
    return pooled


# speedup vs baseline: 1.0260x; 1.0260x over previous
import functools

import jax
import jax.numpy as jnp
from jax import lax
from jax.experimental import pallas as pl
from jax.experimental.pallas import tpu as pltpu
from jax.experimental.pallas import tpu_sc as plsc

D_MODEL = 1024
BATCH = 16
LANES = 16
CHUNKS = D_MODEL // LANES


def _pool_body(hs_hbm, seq_hbm, out_hbm, seq_v, row_v):
    w = lax.axis_index("s")
    pltpu.sync_copy(seq_hbm, seq_v)
    seq = seq_v[...]
    lane = lax.iota(jnp.int32, 16)
    my_idx = jnp.sum(jnp.where(lane <= w, seq, 0)) - 1
    pltpu.sync_copy(hs_hbm.at[my_idx], row_v)

    def _ss_body(j, acc):
        base = j * (8 * LANES)
        for u in range(8):
            ch = row_v[pl.ds(base + u * LANES, LANES)]
            acc = acc + ch * ch
        return acc

    acc = lax.fori_loop(
        0, CHUNKS // 8, _ss_body, jnp.zeros((LANES,), jnp.float32)
    )
    tot = jnp.maximum(jnp.sum(acc), 1e-24)
    t = jnp.full((LANES,), tot, dtype=jnp.float32)
    bits = lax.bitcast_convert_type(t, jnp.int32)
    bits = 0x5F3759DF - lax.shift_right_arithmetic(bits, 1)
    y = lax.bitcast_convert_type(bits, jnp.float32)
    for _ in range(3):
        y = y * (1.5 - 0.5 * t * y * y)

    def _scale_body(j, carry):
        base = j * (8 * LANES)
        for u in range(8):
            sl = pl.ds(base + u * LANES, LANES)
            row_v[sl] = row_v[sl] * y
        return carry

    lax.fori_loop(0, CHUNKS // 8, _scale_body, 0)
    pltpu.sync_copy(row_v, out_hbm.at[w])


def kernel(hidden_states, extend_seq_lens):
    seq = extend_seq_lens.astype(jnp.int32)
    pooled = functools.partial(
        pl.kernel,
        mesh=plsc.VectorSubcoreMesh(
            core_axis_name="c", subcore_axis_name="s", num_cores=1
        ),
        out_type=jax.ShapeDtypeStruct((BATCH, D_MODEL), jnp.float32),
        scratch_types=[
            pltpu.VMEM((BATCH,), jnp.int32),
            pltpu.VMEM((D_MODEL,), jnp.float32),
        ],
        compiler_params=pltpu.CompilerParams(needs_layout_passes=False),
    )(_pool_body)(hidden_states, seq)
    return pooled
